# baseline (speedup 1.0000x reference)
import functools

import jax
import jax.numpy as jnp
from jax import lax
from jax.experimental import pallas as pl
from jax.experimental.pallas import tpu as pltpu
from jax.experimental.pallas import tpu_sc as plsc

_NC = 2
_NS = 16
_CH = 128


def _make_sc_gather(B, D_G, D_M):
    NW = _NC * _NS
    bpw = B // NW
    nch = bpw // _CH

    mesh = plsc.VectorSubcoreMesh(core_axis_name="c", subcore_axis_name="s")

    @functools.partial(
        pl.kernel,
        out_type=[
            jax.ShapeDtypeStruct((B, 128), jnp.float32),
            jax.ShapeDtypeStruct((B, 128), jnp.float32),
        ],
        mesh=mesh,
        compiler_params=pltpu.CompilerParams(use_tc_tiling_on_sc=False),
        scratch_types=[
            pltpu.VMEM((nch, _CH), jnp.int32),
            pltpu.VMEM((nch, _CH), jnp.int32),
            pltpu.VMEM((2, _CH, 128), jnp.float32),
            pltpu.VMEM((2, _CH, 128), jnp.float32),
            pltpu.SemaphoreType.DMA,
            pltpu.SemaphoreType.DMA,
        ],
    )
    def gather_k(uidx_h, iidx_h, pu_h, pi_h,
                 x_o, g_o,
                 uidx_v, iidx_v, pu_v, pi_v, gsem, wsem):
        wid = lax.axis_index("s") * _NC + lax.axis_index("c")
        base = wid * bpw
        for j in range(nch):
            pltpu.sync_copy(uidx_h.at[pl.ds(base + j * _CH, _CH)], uidx_v.at[j])
            pltpu.sync_copy(iidx_h.at[pl.ds(base + j * _CH, _CH)], iidx_v.at[j])
        gu = pltpu.async_copy(pu_h.at[uidx_v.at[0]], pu_v.at[0], gsem)
        gi = pltpu.async_copy(pi_h.at[iidx_v.at[0]], pi_v.at[0], gsem)
        for j in range(nch):
            gu.wait()
            gi.wait()
            if j + 1 < nch:
                gu = pltpu.async_copy(
                    pu_h.at[uidx_v.at[j + 1]], pu_v.at[(j + 1) % 2], gsem)
                gi = pltpu.async_copy(
                    pi_h.at[iidx_v.at[j + 1]], pi_v.at[(j + 1) % 2], gsem)
            sl = pl.ds(base + j * _CH, _CH)
            b = j % 2
            writes = [
                pltpu.async_copy(
                    pu_v.at[b, slice(None), pl.ds(0, D_M)],
                    x_o.at[sl, pl.ds(0, D_M)], wsem),
                pltpu.async_copy(
                    pi_v.at[b, slice(None), pl.ds(0, D_M)],
                    x_o.at[sl, pl.ds(D_M, D_M)], wsem),
                pltpu.async_copy(
                    pu_v.at[b, slice(None), pl.ds(D_M, D_G)],
                    g_o.at[sl, pl.ds(0, D_G)], wsem),
                pltpu.async_copy(
                    pi_v.at[b, slice(None), pl.ds(D_M, D_G)],
                    g_o.at[sl, pl.ds(D_G, D_G)], wsem),
            ]
            for w in writes:
                w.wait()

    return gather_k


def _mlp_body(D_G, x_r, g_r, w1_r, b1_r, w2_r, b2_r,
              w3_r, b3_r, wpg_r, wpm_r, bp_r, o_r):
    dn = (((1,), (1,)), ((), ()))
    h = jnp.maximum(
        lax.dot_general(x_r[...], w1_r[...], dn, preferred_element_type=jnp.float32)
        + b1_r[...], 0.0)
    h = jnp.maximum(
        lax.dot_general(h, w2_r[...], dn, preferred_element_type=jnp.float32)
        + b2_r[...], 0.0)
    h = jnp.maximum(
        lax.dot_general(h, w3_r[...], dn, preferred_element_type=jnp.float32)
        + b3_r[...], 0.0)
    g = g_r[...]
    gmf = g[:, :D_G] * g[:, D_G:2 * D_G]
    logit = (jnp.sum(gmf * wpg_r[...], axis=1)
             + jnp.sum(h * wpm_r[...], axis=1) + bp_r[0, 0])
    o_r[...] = jax.nn.sigmoid(logit)


def _make_tc_mlp(B, D_G, H1, H2, H3, BLK=512):
    nblk = B // BLK
    full = lambda r, c: pl.BlockSpec((r, c), lambda i: (0, 0))
    return pl.pallas_call(
        functools.partial(_mlp_body, D_G),
        grid=(nblk,),
        in_specs=[
            pl.BlockSpec((BLK, 128), lambda i: (i, 0)),
            pl.BlockSpec((BLK, 128), lambda i: (i, 0)),
            full(H1, 128),
            full(1, H1),
            full(H2, H1),
            full(1, H2),
            full(H3, H2),
            full(1, H3),
            full(1, D_G),
            full(1, H3),
            pl.BlockSpec(memory_space=pltpu.SMEM),
        ],
        out_specs=pl.BlockSpec((BLK,), lambda i: (i,)),
        out_shape=jax.ShapeDtypeStruct((B,), jnp.float32),
    )


def kernel(user_indices, item_indices, user_gmf, item_gmf, user_mlp, item_mlp,
           W1, b1, W2, b2, W3, b3, Wp, bp):
    B = user_indices.shape[0]
    D_G = user_gmf.shape[1]
    D_M = user_mlp.shape[1]
    H1, H2, H3 = W1.shape[0], W2.shape[0], W3.shape[0]
    V = user_gmf.shape[0]
    pad = 128 - D_M - D_G

    ui = user_indices.astype(jnp.int32)
    ii = item_indices.astype(jnp.int32)
    z = jnp.zeros((V, pad), jnp.float32)
    pu = jnp.concatenate([user_mlp, user_gmf, z], axis=1)
    pi = jnp.concatenate([item_mlp, item_gmf, z], axis=1)
    xcat, gcat = _make_sc_gather(B, D_G, D_M)(ui, ii, pu, pi)

    mlp = _make_tc_mlp(B, D_G, H1, H2, H3)
    return mlp(xcat, gcat,
               W1, b1.reshape(1, H1),
               W2, b2.reshape(1, H2), W3, b3.reshape(1, H3),
               Wp[:, :D_G], Wp[:, D_G:], bp.reshape(1, 1))


# speedup vs baseline: 1.1682x; 1.1682x over previous
import functools

import jax
import jax.numpy as jnp
from jax import lax
from jax.experimental import pallas as pl
from jax.experimental.pallas import tpu as pltpu
from jax.experimental.pallas import tpu_sc as plsc

_NC = 2
_NS = 16
_CH = 128


def _make_sc_gather_pair(B, D_A, D_B):
    NW = _NC * _NS
    bpw = B // NW
    nch = bpw // _CH

    mesh = plsc.VectorSubcoreMesh(core_axis_name="c", subcore_axis_name="s")

    @functools.partial(
        pl.kernel,
        out_type=[
            jax.ShapeDtypeStruct((B, D_A), jnp.float32),
            jax.ShapeDtypeStruct((B, D_B), jnp.float32),
        ],
        mesh=mesh,
        scratch_types=[
            pltpu.VMEM((bpw,), jnp.int32),
            pltpu.VMEM((bpw,), jnp.int32),
            pltpu.VMEM((2, _CH, D_A), jnp.float32),
            pltpu.VMEM((2, _CH, D_B), jnp.float32),
            pltpu.SemaphoreType.DMA,
            pltpu.SemaphoreType.DMA,
        ],
    )
    def gather_k(uidx_h, iidx_h, a_h, b_h,
                 a_o, b_o,
                 uidx_v, iidx_v, a_v, b_v, gsem, wsem):
        wid = lax.axis_index("s") * _NC + lax.axis_index("c")
        base = wid * bpw
        pltpu.sync_copy(uidx_h.at[pl.ds(base, bpw)], uidx_v)
        pltpu.sync_copy(iidx_h.at[pl.ds(base, bpw)], iidx_v)

        def fetch_chunk(c, buf):
            def fetch(g, _):
                xu = uidx_v[pl.ds(c * _CH + g * 16, 16)]
                xi = iidx_v[pl.ds(c * _CH + g * 16, 16)]
                for k in range(16):
                    i = g * 16 + k
                    pltpu.async_copy(a_h.at[xu[k]], a_v.at[buf, i], gsem)
                    pltpu.async_copy(b_h.at[xi[k]], b_v.at[buf, i], gsem)
                return 0

            lax.fori_loop(0, _CH // 16, fetch, 0)

        def drain_chunk(buf):
            pltpu.make_async_copy(a_h.at[pl.ds(0, _CH)], a_v.at[buf], gsem).wait()
            pltpu.make_async_copy(b_h.at[pl.ds(0, _CH)], b_v.at[buf], gsem).wait()

        fetch_chunk(0, 0)
        for c in range(nch):
            drain_chunk(c % 2)
            if c + 1 < nch:
                fetch_chunk(c + 1, (c + 1) % 2)
            sl = pl.ds(base + c * _CH, _CH)
            wa = pltpu.async_copy(a_v.at[c % 2], a_o.at[sl], wsem)
            wb = pltpu.async_copy(b_v.at[c % 2], b_o.at[sl], wsem)
            wa.wait()
            wb.wait()

    return gather_k


def _mlp_body(D_G, ug_r, ig_r, um_r, im_r, w1u_r, w1i_r, b1_r, w2_r, b2_r,
              w3_r, b3_r, wpg_r, wpm_r, bp_r, o_r):
    dn = (((1,), (1,)), ((), ()))
    h = jnp.maximum(
        lax.dot_general(um_r[...], w1u_r[...], dn, preferred_element_type=jnp.float32)
        + lax.dot_general(im_r[...], w1i_r[...], dn, preferred_element_type=jnp.float32)
        + b1_r[...], 0.0)
    h = jnp.maximum(
        lax.dot_general(h, w2_r[...], dn, preferred_element_type=jnp.float32)
        + b2_r[...], 0.0)
    h = jnp.maximum(
        lax.dot_general(h, w3_r[...], dn, preferred_element_type=jnp.float32)
        + b3_r[...], 0.0)
    gmf = ug_r[...] * ig_r[...]
    logit = (jnp.sum(gmf * wpg_r[...], axis=1)
             + jnp.sum(h * wpm_r[...], axis=1) + bp_r[0, 0])
    o_r[...] = jax.nn.sigmoid(logit)


def _make_tc_mlp(B, D_G, D_M, H1, H2, H3, BLK=512):
    nblk = B // BLK
    full = lambda r, c: pl.BlockSpec((r, c), lambda i: (0, 0))
    return pl.pallas_call(
        functools.partial(_mlp_body, D_G),
        grid=(nblk,),
        in_specs=[
            pl.BlockSpec((BLK, D_G), lambda i: (i, 0)),
            pl.BlockSpec((BLK, D_G), lambda i: (i, 0)),
            pl.BlockSpec((BLK, D_M), lambda i: (i, 0)),
            pl.BlockSpec((BLK, D_M), lambda i: (i, 0)),
            full(H1, D_M),
            full(H1, D_M),
            full(1, H1),
            full(H2, H1),
            full(1, H2),
            full(H3, H2),
            full(1, H3),
            full(1, D_G),
            full(1, H3),
            pl.BlockSpec(memory_space=pltpu.SMEM),
        ],
        out_specs=pl.BlockSpec((BLK,), lambda i: (i,)),
        out_shape=jax.ShapeDtypeStruct((B,), jnp.float32),
    )


def kernel(user_indices, item_indices, user_gmf, item_gmf, user_mlp, item_mlp,
           W1, b1, W2, b2, W3, b3, Wp, bp):
    B = user_indices.shape[0]
    D_G = user_gmf.shape[1]
    D_M = user_mlp.shape[1]
    H1, H2, H3 = W1.shape[0], W2.shape[0], W3.shape[0]

    ui = user_indices.astype(jnp.int32)
    ii = item_indices.astype(jnp.int32)
    ug, ig = _make_sc_gather_pair(B, D_G, D_G)(ui, ii, user_gmf, item_gmf)
    um, im = _make_sc_gather_pair(B, D_M, D_M)(ui, ii, user_mlp, item_mlp)

    mlp = _make_tc_mlp(B, D_G, D_M, H1, H2, H3)
    return mlp(ug, ig, um, im,
               W1[:, :D_M], W1[:, D_M:], b1.reshape(1, H1),
               W2, b2.reshape(1, H2), W3, b3.reshape(1, H3),
               Wp[:, :D_G], Wp[:, D_G:], bp.reshape(1, 1))


# speedup vs baseline: 1.2002x; 1.0273x over previous
import functools

import jax
import jax.numpy as jnp
from jax import lax
from jax.experimental import pallas as pl
from jax.experimental.pallas import tpu as pltpu
from jax.experimental.pallas import tpu_sc as plsc

_NC = 2
_NS = 16
_CH = 128


def _make_sc_gather_pair(B, D_A, D_B):
    NW = _NC * _NS
    bpw = B // NW
    nch = bpw // _CH

    mesh = plsc.VectorSubcoreMesh(core_axis_name="c", subcore_axis_name="s")

    @functools.partial(
        pl.kernel,
        out_type=[
            jax.ShapeDtypeStruct((B, D_A), jnp.float32),
            jax.ShapeDtypeStruct((B, D_B), jnp.float32),
        ],
        mesh=mesh,
        scratch_types=[
            pltpu.VMEM((bpw,), jnp.int32),
            pltpu.VMEM((bpw,), jnp.int32),
            pltpu.VMEM((2, _CH, D_A), jnp.float32),
            pltpu.VMEM((2, _CH, D_B), jnp.float32),
            pltpu.SemaphoreType.DMA,
            pltpu.SemaphoreType.DMA,
        ],
    )
    def gather_k(uidx_h, iidx_h, a_h, b_h,
                 a_o, b_o,
                 uidx_v, iidx_v, a_v, b_v, gsem, wsem):
        wid = lax.axis_index("s") * _NC + lax.axis_index("c")
        base = wid * bpw
        pltpu.sync_copy(uidx_h.at[pl.ds(base, bpw)], uidx_v)
        pltpu.sync_copy(iidx_h.at[pl.ds(base, bpw)], iidx_v)

        def fetch_chunk(c, buf):
            def fetch(g, _):
                xu = uidx_v[pl.ds(c * _CH + g * 16, 16)]
                xi = iidx_v[pl.ds(c * _CH + g * 16, 16)]
                for k in range(16):
                    i = g * 16 + k
                    pltpu.async_copy(a_h.at[xu[k]], a_v.at[buf, i], gsem)
                    pltpu.async_copy(b_h.at[xi[k]], b_v.at[buf, i], gsem)
                return 0

            lax.fori_loop(0, _CH // 16, fetch, 0)

        def drain_chunk(buf):
            pltpu.make_async_copy(a_h.at[pl.ds(0, _CH)], a_v.at[buf], gsem).wait()
            pltpu.make_async_copy(b_h.at[pl.ds(0, _CH)], b_v.at[buf], gsem).wait()

        fetch_chunk(0, 0)
        for c in range(nch):
            drain_chunk(c % 2)
            if c + 1 < nch:
                fetch_chunk(c + 1, (c + 1) % 2)
            sl = pl.ds(base + c * _CH, _CH)
            wa = pltpu.async_copy(a_v.at[c % 2], a_o.at[sl], wsem)
            wb = pltpu.async_copy(b_v.at[c % 2], b_o.at[sl], wsem)
            wa.wait()
            wb.wait()

    return gather_k


def _mlp_body(D_G, ug_r, ig_r, um_r, im_r, w1u_r, w1i_r, b1_r, w2_r, b2_r,
              w3_r, b3_r, wpg_r, wpm_r, bp_r, o_r):
    dn = (((1,), (1,)), ((), ()))
    h = jnp.maximum(
        lax.dot_general(um_r[...], w1u_r[...], dn, preferred_element_type=jnp.float32)
        + lax.dot_general(im_r[...], w1i_r[...], dn, preferred_element_type=jnp.float32)
        + b1_r[...], 0.0)
    h = jnp.maximum(
        lax.dot_general(h, w2_r[...], dn, preferred_element_type=jnp.float32)
        + b2_r[...], 0.0)
    h = jnp.maximum(
        lax.dot_general(h, w3_r[...], dn, preferred_element_type=jnp.float32)
        + b3_r[...], 0.0)
    gmf = ug_r[...] * ig_r[...]
    logit = (jnp.sum(gmf * wpg_r[...], axis=1)
             + jnp.sum(h * wpm_r[...], axis=1) + bp_r[0, 0])
    o_r[...] = jax.nn.sigmoid(logit)


def _make_tc_mlp(B, D_G, D_M, H1, H2, H3, BLK=2048):
    nblk = B // BLK
    full = lambda r, c: pl.BlockSpec((r, c), lambda i: (0, 0))
    return pl.pallas_call(
        functools.partial(_mlp_body, D_G),
        grid=(nblk,),
        in_specs=[
            pl.BlockSpec((BLK, D_G), lambda i: (i, 0)),
            pl.BlockSpec((BLK, D_G), lambda i: (i, 0)),
            pl.BlockSpec((BLK, D_M), lambda i: (i, 0)),
            pl.BlockSpec((BLK, D_M), lambda i: (i, 0)),
            full(H1, D_M),
            full(H1, D_M),
            full(1, H1),
            full(H2, H1),
            full(1, H2),
            full(H3, H2),
            full(1, H3),
            full(1, D_G),
            full(1, H3),
            pl.BlockSpec(memory_space=pltpu.SMEM),
        ],
        out_specs=pl.BlockSpec((BLK,), lambda i: (i,)),
        out_shape=jax.ShapeDtypeStruct((B,), jnp.float32),
    )


def kernel(user_indices, item_indices, user_gmf, item_gmf, user_mlp, item_mlp,
           W1, b1, W2, b2, W3, b3, Wp, bp):
    B = user_indices.shape[0]
    D_G = user_gmf.shape[1]
    D_M = user_mlp.shape[1]
    H1, H2, H3 = W1.shape[0], W2.shape[0], W3.shape[0]

    ui = user_indices.astype(jnp.int32)
    ii = item_indices.astype(jnp.int32)
    ug, ig = _make_sc_gather_pair(B, D_G, D_G)(ui, ii, user_gmf, item_gmf)
    um, im = _make_sc_gather_pair(B, D_M, D_M)(ui, ii, user_mlp, item_mlp)

    mlp = _make_tc_mlp(B, D_G, D_M, H1, H2, H3)
    return mlp(ug, ig, um, im,
               W1[:, :D_M], W1[:, D_M:], b1.reshape(1, H1),
               W2, b2.reshape(1, H2), W3, b3.reshape(1, H3),
               Wp[:, :D_G], Wp[:, D_G:], bp.reshape(1, 1))
